# baseline (speedup 1.0000x reference)
import functools

import jax
import jax.numpy as jnp
from jax import lax
from jax.experimental import pallas as pl
from jax.experimental.pallas import tpu as pltpu
from jax.experimental.pallas import tpu_sc as plsc

NUM_SEGMENTS = 16
D_MODEL = 1024

_INFO = plsc.get_sparse_core_info()
_NC, _NS, _L = _INFO.num_cores, _INFO.num_subcores, _INFO.num_lanes
_NW = _NC * _NS

_B = 4 * 4096
_BPW = _B // _NW
_NG = _BPW // _L


@functools.partial(
    pl.kernel,
    mesh=plsc.VectorSubcoreMesh(core_axis_name="c", subcore_axis_name="s"),
    out_type=jax.ShapeDtypeStruct((_B, D_MODEL), jnp.float32),
    scratch_types=[
        pltpu.VMEM_SHARED((NUM_SEGMENTS, D_MODEL), jnp.float32),
        pltpu.VMEM((_BPW,), jnp.int32),
        pltpu.SemaphoreType.DMA,
    ],
)
def _sc_lookup(seg_hbm, table_hbm, out_hbm, table_sp, idx_v, wsem):
    sid = lax.axis_index("s")
    wid = sid * _NC + lax.axis_index("c")
    base = wid * _BPW

    @pl.when(sid == 0)
    def _stage_table():
        pltpu.sync_copy(table_hbm, table_sp)

    pltpu.sync_copy(seg_hbm.at[pl.ds(base, _BPW)], idx_v)
    plsc.subcore_barrier()

    def issue_group(g, _):
        idxs = idx_v[pl.ds(g * _L, _L)]
        row = base + g * _L
        for l in range(_L):
            pltpu.async_copy(table_sp.at[idxs[l]], out_hbm.at[row + l], wsem)
        return 0

    lax.fori_loop(0, _NG, issue_group, 0)

    def drain_group(g, _):
        pltpu.make_async_copy(
            table_sp, out_hbm.at[pl.ds(base + g * _L, _L)], wsem
        ).wait()
        return 0

    lax.fori_loop(0, _NG, drain_group, 0)


def kernel(segment_ids, table):
    seg_flat = segment_ids.reshape(-1).astype(jnp.int32)
    out = _sc_lookup(seg_flat, table)
    return out.reshape(segment_ids.shape + (D_MODEL,))


# speedup vs baseline: 1.1367x; 1.1367x over previous
import functools

import jax
import jax.numpy as jnp
from jax import lax
from jax.experimental import pallas as pl
from jax.experimental.pallas import tpu as pltpu
from jax.experimental.pallas import tpu_sc as plsc

NUM_SEGMENTS = 16
D_MODEL = 1024

_INFO = plsc.get_sparse_core_info()
_NC, _NS, _L = _INFO.num_cores, _INFO.num_subcores, _INFO.num_lanes
_NW = _NC * _NS

_B = 4 * 4096
_BPW = _B // _NW
_NG = _BPW // _L


@functools.partial(
    pl.kernel,
    mesh=plsc.VectorSubcoreMesh(core_axis_name="c", subcore_axis_name="s"),
    out_type=jax.ShapeDtypeStruct((_B, D_MODEL), jnp.float32),
    scratch_types=[
        pltpu.VMEM((NUM_SEGMENTS, D_MODEL), jnp.float32),
        pltpu.VMEM((_BPW,), jnp.int32),
        pltpu.SemaphoreType.DMA,
    ],
)
def _sc_lookup(seg_hbm, table_hbm, out_hbm, table_v, idx_v, wsem):
    wid = lax.axis_index("s") * _NC + lax.axis_index("c")
    base = wid * _BPW
    pltpu.sync_copy(table_hbm, table_v)
    pltpu.sync_copy(seg_hbm.at[pl.ds(base, _BPW)], idx_v)

    def issue_group(g, _):
        idxs = idx_v[pl.ds(g * _L, _L)]
        row = base + g * _L
        for l in range(_L):
            pltpu.async_copy(table_v.at[idxs[l]], out_hbm.at[row + l], wsem)
        return 0

    lax.fori_loop(0, _NG, issue_group, 0)

    def drain_group(g, _):
        pltpu.make_async_copy(
            table_v, out_hbm.at[pl.ds(base + g * _L, _L)], wsem
        ).wait()
        return 0

    lax.fori_loop(0, _NG, drain_group, 0)


def kernel(segment_ids, table):
    seg_flat = segment_ids.reshape(-1).astype(jnp.int32)
    out = _sc_lookup(seg_flat, table)
    return out.reshape(segment_ids.shape + (D_MODEL,))
